# baseline (speedup 1.0000x reference)
import functools

import jax
import jax.numpy as jnp
from jax import lax
from jax.experimental import pallas as pl
from jax.experimental.pallas import tpu as pltpu

NN = 10000
NP = 10240
HH = 512
BLK = 256
NBLK = NP // BLK


def _mlp0_body(x_ref, p_ref, w1_ref, b1_ref, w2_ref, b2_ref, out_ref):
    u = x_ref[...] + p_ref[0] + p_ref[1]
    t = jnp.maximum(jnp.dot(u, w1_ref[...], preferred_element_type=jnp.float32) + b1_ref[...], 0.0)
    v = jnp.maximum(jnp.dot(t, w2_ref[...], preferred_element_type=jnp.float32) + b2_ref[...], 0.0)
    for c in range(4):
        out_ref[c] = v[:, c * 128:(c + 1) * 128]


def _mlp_body(h_ref, a_ref, w1_ref, b1_ref, w2_ref, b2_ref, out_ref):
    u = jnp.concatenate([h_ref[c] + a_ref[c] for c in range(4)], axis=-1)
    t = jnp.maximum(jnp.dot(u, w1_ref[...], preferred_element_type=jnp.float32) + b1_ref[...], 0.0)
    v = jnp.maximum(jnp.dot(t, w2_ref[...], preferred_element_type=jnp.float32) + b2_ref[...], 0.0)
    for c in range(4):
        out_ref[c] = v[:, c * 128:(c + 1) * 128]


def _mlp_final_body(h_ref, a_ref, w1_ref, b1_ref, w2_ref, b2_ref, wc_ref,
                    bc_ref, out_ref, acc_ref):
    i = pl.program_id(0)
    u = jnp.concatenate([h_ref[c] + a_ref[c] for c in range(4)], axis=-1)
    t = jnp.maximum(jnp.dot(u, w1_ref[...], preferred_element_type=jnp.float32) + b1_ref[...], 0.0)
    v = jnp.maximum(jnp.dot(t, w2_ref[...], preferred_element_type=jnp.float32) + b2_ref[...], 0.0)
    rows = i * BLK + lax.broadcasted_iota(jnp.int32, (BLK, 1), 0)
    v = jnp.where(rows < NN, v, 0.0)

    @pl.when(i == 0)
    def _():
        acc_ref[...] = jnp.zeros_like(acc_ref)

    acc_ref[...] += jnp.sum(v, axis=0, keepdims=True)

    @pl.when(i == NBLK - 1)
    def _():
        pooled = acc_ref[...] * (1.0 / NN)
        out_ref[...] = jnp.dot(pooled, wc_ref[...], preferred_element_type=jnp.float32) + bc_ref[...]


_W_SPEC = pl.BlockSpec((HH, HH), lambda i: (0, 0))
_B_SPEC = pl.BlockSpec((1, HH), lambda i: (0, 0))
_H4_SPEC = pl.BlockSpec((4, BLK, 128), lambda i: (0, i, 0))


def _mlp0(x16, p16, W1p, b1, W2, b2):
    return pl.pallas_call(
        _mlp0_body,
        grid=(NBLK,),
        in_specs=[
            pl.BlockSpec((BLK, 16), lambda i: (i, 0)),
            pl.BlockSpec((2, BLK, 16), lambda i: (0, i, 0)),
            pl.BlockSpec((16, HH), lambda i: (0, 0)),
            _B_SPEC, _W_SPEC, _B_SPEC,
        ],
        out_specs=_H4_SPEC,
        out_shape=jax.ShapeDtypeStruct((4, NP, 128), jnp.float32),
    )(x16, p16, W1p, b1.reshape(1, HH), W2, b2.reshape(1, HH))


def _mlp(h4, a4, W1, b1, W2, b2):
    return pl.pallas_call(
        _mlp_body,
        grid=(NBLK,),
        in_specs=[_H4_SPEC, _H4_SPEC, _W_SPEC, _B_SPEC, _W_SPEC, _B_SPEC],
        out_specs=_H4_SPEC,
        out_shape=jax.ShapeDtypeStruct((4, NP, 128), jnp.float32),
    )(h4, a4, W1, b1.reshape(1, HH), W2, b2.reshape(1, HH))


def _mlp_final(h4, a4, W1, b1, W2, b2, Wc, bc):
    return pl.pallas_call(
        _mlp_final_body,
        grid=(NBLK,),
        in_specs=[
            _H4_SPEC, _H4_SPEC, _W_SPEC, _B_SPEC, _W_SPEC, _B_SPEC,
            pl.BlockSpec((HH, 1), lambda i: (0, 0)),
            pl.BlockSpec((1, 1), lambda i: (0, 0)),
        ],
        out_specs=pl.BlockSpec((1, 1), lambda i: (0, 0)),
        out_shape=jax.ShapeDtypeStruct((1, 1), jnp.float32),
        scratch_shapes=[pltpu.VMEM((1, HH), jnp.float32)],
    )(h4, a4, W1, b1.reshape(1, HH), W2, b2.reshape(1, HH), Wc, bc.reshape(1, 1))


def _flatten_h4(h4):
    return jnp.transpose(h4, (1, 0, 2)).reshape(NP, HH)


def _chunk_h(h):
    return jnp.transpose(h.reshape(NP, 4, 128), (1, 0, 2))


def _agg_xla(h_flat, src, dst):
    msgs = jnp.take(h_flat, src, axis=0)
    return jax.ops.segment_sum(msgs, dst, num_segments=NP)


def kernel(x, edge_index, W1_0, b1_0, W2_0, b2_0, W1_1, b1_1, W2_1, b2_1,
           W1_2, b1_2, W2_2, b2_2, Wc, bc):
    src = edge_index[0]
    dst = edge_index[1]

    x16 = jnp.zeros((NP, 16), jnp.float32).at[:NN, :2].set(x)
    W1_0p = jnp.zeros((16, HH), jnp.float32).at[:2, :].set(W1_0)

    agg0 = _agg_xla(x16, src, dst)
    p16 = jnp.stack([agg0, jnp.zeros_like(agg0)])
    h1 = _mlp0(x16, p16, W1_0p, b1_0, W2_0, b2_0)

    a1 = _chunk_h(_agg_xla(_flatten_h4(h1), src, dst))
    h2 = _mlp(h1, a1, W1_1, b1_1, W2_1, b2_1)

    a2 = _chunk_h(_agg_xla(_flatten_h4(h2), src, dst))
    out = _mlp_final(h2, a2, W1_2, b1_2, W2_2, b2_2, Wc, bc)
    return out.reshape(-1)


# speedup vs baseline: 7.7024x; 7.7024x over previous
import jax
import jax.numpy as jnp
from jax import lax
from jax.experimental import pallas as pl
from jax.experimental.pallas import tpu as pltpu
from jax.experimental.pallas import tpu_sc as plsc

NN = 10000
NP = 10240
HH = 512
BLK = 256
NBLK = NP // BLK

EE = 160000
ECAP = 163840
EPT = ECAP // 16
NB128 = EPT // 128
HB128 = NB128 // 2
EPT16 = ECAP // 32
NB16 = EPT16 // 128
RPT = NP // 16


def _mlp0_body(x_ref, p_ref, w1_ref, b1_ref, w2_ref, b2_ref, out_ref):
    u = x_ref[...] + p_ref[0] + p_ref[1]
    t = jnp.maximum(jnp.dot(u, w1_ref[...], preferred_element_type=jnp.float32) + b1_ref[...], 0.0)
    v = jnp.maximum(jnp.dot(t, w2_ref[...], preferred_element_type=jnp.float32) + b2_ref[...], 0.0)
    for c in range(4):
        out_ref[c] = v[:, c * 128:(c + 1) * 128]


def _mlp_body(h_ref, a_ref, w1_ref, b1_ref, w2_ref, b2_ref, out_ref):
    u = jnp.concatenate([h_ref[c] + a_ref[c] for c in range(4)], axis=-1)
    t = jnp.maximum(jnp.dot(u, w1_ref[...], preferred_element_type=jnp.float32) + b1_ref[...], 0.0)
    v = jnp.maximum(jnp.dot(t, w2_ref[...], preferred_element_type=jnp.float32) + b2_ref[...], 0.0)
    for c in range(4):
        out_ref[c] = v[:, c * 128:(c + 1) * 128]


def _mlp_final_body(h_ref, a_ref, w1_ref, b1_ref, w2_ref, b2_ref, wc_ref,
                    bc_ref, out_ref, acc_ref):
    i = pl.program_id(0)
    u = jnp.concatenate([h_ref[c] + a_ref[c] for c in range(4)], axis=-1)
    t = jnp.maximum(jnp.dot(u, w1_ref[...], preferred_element_type=jnp.float32) + b1_ref[...], 0.0)
    v = jnp.maximum(jnp.dot(t, w2_ref[...], preferred_element_type=jnp.float32) + b2_ref[...], 0.0)
    rows = i * BLK + lax.broadcasted_iota(jnp.int32, (BLK, 1), 0)
    v = jnp.where(rows < NN, v, 0.0)

    @pl.when(i == 0)
    def _():
        acc_ref[...] = jnp.zeros_like(acc_ref)

    acc_ref[...] += jnp.sum(v, axis=0, keepdims=True)

    @pl.when(i == NBLK - 1)
    def _():
        pooled = acc_ref[...] * (1.0 / NN)
        out_ref[...] = jnp.dot(pooled, wc_ref[...], preferred_element_type=jnp.float32) + bc_ref[...]


_W_SPEC = pl.BlockSpec((HH, HH), lambda i: (0, 0))
_B_SPEC = pl.BlockSpec((1, HH), lambda i: (0, 0))
_H4_SPEC = pl.BlockSpec((4, BLK, 128), lambda i: (0, i, 0))


def _mlp0(x16, p16, W1p, b1, W2, b2):
    return pl.pallas_call(
        _mlp0_body,
        grid=(NBLK,),
        in_specs=[
            pl.BlockSpec((BLK, 16), lambda i: (i, 0)),
            pl.BlockSpec((2, BLK, 16), lambda i: (0, i, 0)),
            pl.BlockSpec((16, HH), lambda i: (0, 0)),
            _B_SPEC, _W_SPEC, _B_SPEC,
        ],
        out_specs=_H4_SPEC,
        out_shape=jax.ShapeDtypeStruct((4, NP, 128), jnp.float32),
    )(x16, p16, W1p, b1.reshape(1, HH), W2, b2.reshape(1, HH))


def _mlp(h4, a4, W1, b1, W2, b2):
    return pl.pallas_call(
        _mlp_body,
        grid=(NBLK,),
        in_specs=[_H4_SPEC, _H4_SPEC, _W_SPEC, _B_SPEC, _W_SPEC, _B_SPEC],
        out_specs=_H4_SPEC,
        out_shape=jax.ShapeDtypeStruct((4, NP, 128), jnp.float32),
    )(h4, a4, W1, b1.reshape(1, HH), W2, b2.reshape(1, HH))


def _mlp_final(h4, a4, W1, b1, W2, b2, Wc, bc):
    return pl.pallas_call(
        _mlp_final_body,
        grid=(NBLK,),
        in_specs=[
            _H4_SPEC, _H4_SPEC, _W_SPEC, _B_SPEC, _W_SPEC, _B_SPEC,
            pl.BlockSpec((HH, 1), lambda i: (0, 0)),
            pl.BlockSpec((1, 1), lambda i: (0, 0)),
        ],
        out_specs=pl.BlockSpec((1, 1), lambda i: (0, 0)),
        out_shape=jax.ShapeDtypeStruct((1, 1), jnp.float32),
        scratch_shapes=[pltpu.VMEM((1, HH), jnp.float32)],
    )(h4, a4, W1, b1.reshape(1, HH), W2, b2.reshape(1, HH), Wc, bc.reshape(1, 1))


_SC_MESH = plsc.VectorSubcoreMesh(core_axis_name="c", subcore_axis_name="s")


def _fill_zeros(zbuf, rows, cols):
    z = jnp.zeros((16,), jnp.float32)
    for r in range(rows):
        for c in range(cols // 16):
            zbuf[r, pl.ds(c * 16, 16)] = z


def _seg128_body(h4f, srcp4, dstp, out, acc, src_v, dst_v, buf0, buf1, zbuf,
                 sem0, sem1):
    sc = lax.axis_index("c")
    tid = lax.axis_index("s")
    _fill_zeros(zbuf, 8, 128)
    zrow = tid * RPT
    bufs = (buf0, buf1)
    sems = (sem0, sem1)
    for ci in range(2):
        chunk = sc * 2 + ci
        for z in range(RPT // 8):
            pltpu.sync_copy(zbuf, acc.at[pl.ds(zrow + z * 8, 8)])
        plsc.subcore_barrier()
        for hf in range(2):
            pltpu.sync_copy(srcp4.at[chunk, tid, pl.ds(hf * HB128, HB128)],
                            src_v)
            pltpu.sync_copy(dstp.at[tid, pl.ds(hf * HB128, HB128)], dst_v)
            prev = pltpu.async_copy(h4f.at[src_v.at[0]], bufs[0], sems[0])
            for b in range(HB128):
                nxt = None
                if b + 1 < HB128:
                    nxt = pltpu.async_copy(h4f.at[src_v.at[b + 1]],
                                           bufs[(b + 1) % 2],
                                           sems[(b + 1) % 2])
                prev.wait()
                pltpu.sync_copy(bufs[b % 2], acc.at[dst_v.at[b]], add=True)
                prev = nxt
        plsc.subcore_barrier()
        pltpu.sync_copy(acc.at[pl.ds(zrow, RPT)],
                        out.at[pl.ds(chunk * NP + zrow, RPT)])


def _seg16_body(h16, srcp, dstp, out, acc, src_v, dst_v, buf0, buf1, zbuf,
                sem0, sem1):
    sc = lax.axis_index("c")
    tid = lax.axis_index("s")
    _fill_zeros(zbuf, 16, 16)
    pltpu.sync_copy(srcp.at[sc, tid], src_v)
    pltpu.sync_copy(dstp.at[sc, tid], dst_v)
    zrow = tid * RPT
    for z in range(RPT // 16):
        pltpu.sync_copy(zbuf, acc.at[pl.ds(zrow + z * 16, 16)])
    plsc.subcore_barrier()
    bufs = (buf0, buf1)
    sems = (sem0, sem1)
    prev = pltpu.async_copy(h16.at[src_v.at[0]], bufs[0], sems[0])
    for b in range(NB16):
        nxt = None
        if b + 1 < NB16:
            nxt = pltpu.async_copy(h16.at[src_v.at[b + 1]],
                                   bufs[(b + 1) % 2], sems[(b + 1) % 2])
        prev.wait()
        pltpu.sync_copy(bufs[b % 2], acc.at[dst_v.at[b]], add=True)
        prev = nxt
    plsc.subcore_barrier()
    pltpu.sync_copy(acc.at[pl.ds(zrow, RPT)], out.at[sc, pl.ds(zrow, RPT)])


_seg128 = pl.kernel(
    _seg128_body,
    out_type=jax.ShapeDtypeStruct((4 * NP, 128), jnp.float32),
    mesh=_SC_MESH,
    scratch_types=[
        pltpu.VMEM_SHARED((NP, 128), jnp.float32),
        pltpu.VMEM((HB128, 128), jnp.int32),
        pltpu.VMEM((HB128, 128), jnp.int32),
        pltpu.VMEM((128, 128), jnp.float32),
        pltpu.VMEM((128, 128), jnp.float32),
        pltpu.VMEM((8, 128), jnp.float32),
        pltpu.SemaphoreType.DMA,
        pltpu.SemaphoreType.DMA,
    ],
)

_seg16 = pl.kernel(
    _seg16_body,
    out_type=jax.ShapeDtypeStruct((2, NP, 16), jnp.float32),
    mesh=_SC_MESH,
    compiler_params=pltpu.CompilerParams(use_tc_tiling_on_sc=False),
    scratch_types=[
        pltpu.VMEM_SHARED((NP, 16), jnp.float32),
        pltpu.VMEM((NB16, 128), jnp.int32),
        pltpu.VMEM((NB16, 128), jnp.int32),
        pltpu.VMEM((128, 16), jnp.float32),
        pltpu.VMEM((128, 16), jnp.float32),
        pltpu.VMEM((16, 16), jnp.float32),
        pltpu.SemaphoreType.DMA,
        pltpu.SemaphoreType.DMA,
    ],
)


def kernel(x, edge_index, W1_0, b1_0, W2_0, b2_0, W1_1, b1_1, W2_1, b2_1,
           W1_2, b1_2, W2_2, b2_2, Wc, bc):
    src = edge_index[0]
    dst = edge_index[1]

    pad = ECAP - EE
    psrc = jnp.concatenate([src, jnp.arange(pad, dtype=jnp.int32) % 9973])
    pdst = jnp.concatenate(
        [dst, NN + jnp.arange(pad, dtype=jnp.int32) % (NP - NN)])

    srcp4 = (psrc.reshape(1, 16, NB128, 128)
             + (jnp.arange(4, dtype=jnp.int32) * NP).reshape(4, 1, 1, 1))
    dstp128 = pdst.reshape(16, NB128, 128)
    srcp16 = psrc.reshape(2, 16, NB16, 128)
    dstp16 = pdst.reshape(2, 16, NB16, 128)

    x16 = jnp.zeros((NP, 16), jnp.float32).at[:NN, :2].set(x)
    W1_0p = jnp.zeros((16, HH), jnp.float32).at[:2, :].set(W1_0)

    p16 = _seg16(x16, srcp16, dstp16)
    h1 = _mlp0(x16, p16, W1_0p, b1_0, W2_0, b2_0)

    a1 = _seg128(h1.reshape(4 * NP, 128), srcp4, dstp128).reshape(4, NP, 128)
    h2 = _mlp(h1, a1, W1_1, b1_1, W2_1, b2_1)

    a2 = _seg128(h2.reshape(4 * NP, 128), srcp4, dstp128).reshape(4, NP, 128)
    out = _mlp_final(h2, a2, W1_2, b1_2, W2_2, b2_2, Wc, bc)
    return out.reshape(-1)
